# baseline (speedup 1.0000x reference)
import jax
import jax.numpy as jnp
from jax import lax
from jax.experimental import pallas as pl
from jax.experimental.pallas import tpu as pltpu
from jax.experimental.pallas import tpu_sc as plsc

N = 10000
D = 128
E = 320000
MAX_ITER = 10
STEP = 0.1 / 1.5
YC = 16
NC = 2
NS = 16
NW = NC * NS
NPT = 320
NP = NW * NPT
TRASH = NPT
CAP = 32768
FLUSH = 1024
ECH = 3200
NECH = E // ECH
EB = 64
ICH = FLUSH // EB


def _filter_kernel():
    mesh = plsc.VectorSubcoreMesh(core_axis_name="c", subcore_axis_name="s",
                                  num_cores=NC, num_subcores=NS)

    def body(src_hbm, dst_hbm, oth_hbm, own_hbm, cnt_hbm,
             sc_buf, dc_buf, so_buf, sw_buf, cnt_buf):
        cid = lax.axis_index("c")
        sid = lax.axis_index("s")
        wid = sid * NC + cid
        lo = wid * NPT
        lanes = lax.iota(jnp.int32, 16)
        maxflush = CAP // FLUSH

        def side(o, idx_own, idx_oth):
            m = jnp.logical_and(idx_own >= lo, idx_own < lo + NPT)
            mi = m.astype(jnp.int32)
            pos = o + plsc.cumsum(mi) - mi
            plsc.store_scatter(so_buf, [pos], idx_oth, mask=m)
            plsc.store_scatter(sw_buf, [pos], idx_own - lo, mask=m)
            pc = plsc.all_reduce_population_count(m)
            return o + pc[0]

        def flush(carry):
            o, nf = carry

            def do_flush(c2):
                o2, nf2 = c2
                nfc = jnp.minimum(nf2, maxflush - 1)
                pltpu.sync_copy(so_buf.at[pl.ds(0, FLUSH)],
                                oth_hbm.at[wid, pl.ds(nfc * FLUSH, FLUSH)])
                pltpu.sync_copy(sw_buf.at[pl.ds(0, FLUSH)],
                                own_hbm.at[wid, pl.ds(nfc * FLUSH, FLUSH)])
                for gg in range(2):
                    idx = FLUSH + gg * 16 + lanes
                    mm = idx < o2
                    va = plsc.load_gather(so_buf, [idx])
                    vb = plsc.load_gather(sw_buf, [idx])
                    plsc.store_scatter(so_buf, [idx - FLUSH], va, mask=mm)
                    plsc.store_scatter(sw_buf, [idx - FLUSH], vb, mask=mm)
                return (o2 - FLUSH, nf2 + 1)

            return lax.cond(o >= FLUSH, do_flush, lambda c2: c2, (o, nf))

        def chunk_body(c, carry):
            pltpu.sync_copy(src_hbm.at[pl.ds(c * ECH, ECH)], sc_buf)
            pltpu.sync_copy(dst_hbm.at[pl.ds(c * ECH, ECH)], dc_buf)

            def grp_body(g, carry2):
                sv = sc_buf[pl.ds(g * 16, 16)]
                dv = dc_buf[pl.ds(g * 16, 16)]
                o, nf = carry2
                o = side(o, dv, sv)
                o = side(o, sv, dv)
                return flush((o, nf))

            return lax.fori_loop(0, ECH // 16, grp_body, carry)

        o, nf = lax.fori_loop(0, NECH, chunk_body, (jnp.int32(0), jnp.int32(0)))

        def pad_body(gg, o_):
            idx = gg * 16 + lanes
            mm = idx >= o_
            plsc.store_scatter(so_buf, [idx], jnp.zeros((16,), jnp.int32),
                               mask=mm)
            plsc.store_scatter(sw_buf, [idx],
                               jnp.full((16,), TRASH, jnp.int32), mask=mm)
            return o_

        lax.fori_loop(0, FLUSH // 16, pad_body, o)
        _, nf = flush((jnp.int32(FLUSH), nf))

        cnt_buf[...] = jnp.broadcast_to(nf * FLUSH, (16,)).astype(jnp.int32)
        pltpu.sync_copy(cnt_buf, cnt_hbm.at[wid])

    return pl.kernel(
        body,
        out_type=[jax.ShapeDtypeStruct((NW, CAP), jnp.int32),
                  jax.ShapeDtypeStruct((NW, CAP), jnp.int32),
                  jax.ShapeDtypeStruct((NW, 16), jnp.int32)],
        mesh=mesh,
        scratch_types=[
            pltpu.VMEM((ECH,), jnp.int32),
            pltpu.VMEM((ECH,), jnp.int32),
            pltpu.VMEM((FLUSH + 64,), jnp.int32),
            pltpu.VMEM((FLUSH + 64,), jnp.int32),
            pltpu.VMEM((16,), jnp.int32),
        ],
        compiler_params=pltpu.CompilerParams(needs_layout_passes=False,
                                             use_tc_tiling_on_sc=False),
    )


def _iter_kernel():
    mesh = plsc.VectorSubcoreMesh(core_axis_name="c", subcore_axis_name="s",
                                  num_cores=NC, num_subcores=NS)

    def body(state_hbm, y_hbm, oth_hbm, own_hbm, cnt_hbm, w_hbm,
             acc_out, d_out,
             acc, dacc, yloc, xo, yo, oth, own, wbuf, cbuf,
             gsem0, gsem1):
        cid = lax.axis_index("c")
        sid = lax.axis_index("s")
        wid = sid * NC + cid
        gsems = [gsem0, gsem1]
        lanes = lax.iota(jnp.int32, 16)
        zero16 = jnp.zeros((16,), jnp.float32)

        def zacc(r, carry):
            for j in range(D // 16):
                acc[r, pl.ds(16 * j, 16)] = zero16
            return carry

        lax.fori_loop(0, NPT + 8, zacc, 0)

        def zd(r, carry):
            dacc[r, pl.ds(0, 16)] = zero16
            return carry

        lax.fori_loop(0, NPT + 8, zd, 0)

        def zy(r, carry):
            yloc[NPT + r, pl.ds(0, 16)] = zero16
            return carry

        lax.fori_loop(0, 8, zy, 0)

        pltpu.sync_copy(y_hbm.at[pl.ds(wid * NPT, NPT)],
                        yloc.at[pl.ds(0, NPT)])
        pltpu.sync_copy(w_hbm, wbuf)
        pltpu.sync_copy(cnt_hbm.at[wid], cbuf)
        cnt = jnp.minimum(cbuf[pl.ds(0, 16)][0], CAP)
        nblk = cnt // EB

        wk_vecs = [plsc.load_gather(wbuf, [jnp.full((16,), k, jnp.int32)])
                   for k in range(5)]

        def refill(chunkid, q):
            pltpu.sync_copy(oth_hbm.at[wid, pl.ds(chunkid * FLUSH, FLUSH)],
                            oth.at[q])
            pltpu.sync_copy(own_hbm.at[wid, pl.ds(chunkid * FLUSH, FLUSH)],
                            own.at[q])

        def idx_slice(b):
            q = (b // ICH) % 2
            r = b % ICH
            return oth.at[q, pl.ds(r * EB, EB)]

        def fire_gathers(b, p):
            orow = idx_slice(b)
            pltpu.async_copy(state_hbm.at[orow], xo.at[p], gsems[p])
            pltpu.async_copy(y_hbm.at[orow], yo.at[p], gsems[p])

        def wait_gathers(p):
            pltpu.make_async_copy(state_hbm.at[oth.at[0, pl.ds(0, EB)]],
                                  xo.at[p], gsems[p]).wait()
            pltpu.make_async_copy(y_hbm.at[oth.at[0, pl.ds(0, EB)]],
                                  yo.at[p], gsems[p]).wait()

        def compute(b, p):
            q = (b // ICH) % 2
            r0 = (b % ICH) * EB
            zcol = jnp.zeros((16,), jnp.int32)
            evs = []
            owns = []
            rowss = []
            for g in range(EB // 16):
                rows = g * 16 + lanes
                ow = own[q, pl.ds(r0 + g * 16, 16)]
                e_acc = jnp.zeros((16,), jnp.float32)
                for k in range(5):
                    col = jnp.full((16,), k, jnp.int32)
                    a = plsc.load_gather(yloc, [ow, col])
                    b_ = plsc.load_gather(yo.at[p], [rows, col])
                    e_acc = e_acc + wk_vecs[k] * jnp.abs(a - b_)
                plsc.addupdate_scatter(dacc, [ow, zcol], e_acc)
                evs.append(e_acc)
                owns.append(ow)
                rowss.append(rows)

            def cbody(col, carry):
                colv = jnp.broadcast_to(col, (16,)).astype(jnp.int32)
                for g in range(EB // 16):
                    v = plsc.load_gather(xo.at[p], [rowss[g], colv])
                    plsc.addupdate_scatter(acc, [owns[g], colv], v * evs[g])
                return carry

            lax.fori_loop(0, D, cbody, 0)

        refill(0, 0)
        fire_gathers(0, 0)

        def pair_body(i, carry):
            for p in range(2):
                b = 2 * i + p
                pp = 1 - p
                nb = b + 1

                @pl.when(jnp.logical_and(nb < nblk, nb % ICH == 0))
                def _():
                    refill(nb // ICH, (nb // ICH) % 2)

                @pl.when(nb < nblk)
                def _():
                    fire_gathers(nb, pp)

                wait_gathers(p)
                compute(b, p)
            return carry

        lax.fori_loop(0, nblk // 2, pair_body, 0)

        pltpu.sync_copy(acc.at[pl.ds(0, NPT)],
                        acc_out.at[pl.ds(wid * NPT, NPT)])
        pltpu.sync_copy(dacc.at[pl.ds(0, NPT)],
                        d_out.at[pl.ds(wid * NPT, NPT)])

    return pl.kernel(
        body,
        out_type=[jax.ShapeDtypeStruct((NP, D), jnp.float32),
                  jax.ShapeDtypeStruct((NP, 16), jnp.float32)],
        mesh=mesh,
        scratch_types=[
            pltpu.VMEM((NPT + 8, D), jnp.float32),
            pltpu.VMEM((NPT + 8, 16), jnp.float32),
            pltpu.VMEM((NPT + 8, YC), jnp.float32),
            pltpu.VMEM((2, EB, D), jnp.float32),
            pltpu.VMEM((2, EB, YC), jnp.float32),
            pltpu.VMEM((2, FLUSH), jnp.int32),
            pltpu.VMEM((2, FLUSH), jnp.int32),
            pltpu.VMEM((16,), jnp.float32),
            pltpu.VMEM((16,), jnp.int32),
            pltpu.SemaphoreType.DMA,
            pltpu.SemaphoreType.DMA,
        ],
        compiler_params=pltpu.CompilerParams(needs_layout_passes=False,
                                             use_tc_tiling_on_sc=False),
    )


RB = 1024


def _proj_body(s_ref, w_ref, y_ref):
    y_ref[...] = jnp.dot(s_ref[...], w_ref[...],
                         preferred_element_type=jnp.float32)


def _update_body(s_ref, a_ref, d_ref, w_ref, o_ref, y_ref):
    d = d_ref[...][:, 0:1]
    ns = s_ref[...] - STEP * (a_ref[...] - d * s_ref[...])
    o_ref[...] = ns
    y_ref[...] = jnp.dot(ns, w_ref[...], preferred_element_type=jnp.float32)


def _make_tc_kernels():
    grid = (NP // RB,)
    s_spec = pl.BlockSpec((RB, D), lambda i: (i, 0))
    w_spec = pl.BlockSpec((D, YC), lambda i: (0, 0))
    y_spec = pl.BlockSpec((RB, YC), lambda i: (i, 0))
    d_spec = pl.BlockSpec((RB, 16), lambda i: (i, 0))
    proj = pl.pallas_call(
        _proj_body,
        grid=grid,
        in_specs=[s_spec, w_spec],
        out_specs=y_spec,
        out_shape=jax.ShapeDtypeStruct((NP, YC), jnp.float32),
    )
    update = pl.pallas_call(
        _update_body,
        grid=grid,
        in_specs=[s_spec, s_spec, d_spec, w_spec],
        out_specs=[s_spec, y_spec],
        out_shape=[jax.ShapeDtypeStruct((NP, D), jnp.float32),
                   jax.ShapeDtypeStruct((NP, YC), jnp.float32)],
    )
    return proj, update


def kernel(x, W, bobot, edge_index):
    w = jax.nn.softmax(bobot)
    w16 = jnp.zeros((16,), jnp.float32).at[:5].set(w)
    Wp = jnp.zeros((D, YC), jnp.float32).at[:, :5].set(W)
    src = edge_index[0]
    dst = edge_index[1]

    filt = _filter_kernel()
    step = _iter_kernel()
    proj, update = _make_tc_kernels()

    oth, own, cnt = filt(src, dst)

    state = jnp.zeros((NP, D), jnp.float32).at[:N].set(x)
    Y = proj(state, Wp)
    for _ in range(MAX_ITER):
        acc, dv = step(state, Y, oth, own, cnt, w16)
        state, Y = update(state, acc, dv, Wp)
    return state[:N]


# speedup vs baseline: 7.6690x; 7.6690x over previous
import functools

import jax
import jax.numpy as jnp
from jax import lax
from jax.experimental import pallas as pl
from jax.experimental.pallas import tpu as pltpu
from jax.experimental.pallas import tpu_sc as plsc

N = 10000
D = 128
E = 320000
MAX_ITER = 10
STEP = 0.1 / 1.5
YC = 16
NC = 2
NS = 16
NW = NC * NS
EPW = E // NW
B = 80
NBLK = EPW // B
NROWCHUNKS = N // B
_NCHUNK_CEIL = -(-NROWCHUNKS // NS)


def _sc_step_kernel():
    mesh = plsc.VectorSubcoreMesh(core_axis_name="c", subcore_axis_name="s",
                                  num_cores=NC, num_subcores=NS)

    def body(state_hbm, y_hbm, src_hbm, dst_hbm, w_hbm, out_hbm,
             acc, xs, xd, ys, yd, en, sidx, didx, wbuf, gsem, ssem):
        cid = lax.axis_index("c")
        sid = lax.axis_index("s")
        wid = sid * NC + cid

        zero16 = jnp.zeros((16,), jnp.float32)

        def zbody(r, carry):
            for j in range(D // 16):
                xs[r, pl.ds(16 * j, 16)] = zero16
            return carry

        lax.fori_loop(0, B, zbody, 0)

        for k in range(_NCHUNK_CEIL):
            chunk = sid + k * NS

            @pl.when(chunk < NROWCHUNKS)
            def _():
                pltpu.sync_copy(xs, acc.at[pl.ds(chunk * B, B)])

        pltpu.sync_copy(w_hbm, wbuf)
        plsc.subcore_barrier()

        lanes = lax.iota(jnp.int32, 16)
        wk_vecs = [plsc.load_gather(wbuf, [jnp.full((16,), k, jnp.int32)])
                   for k in range(5)]

        def blk_body(blk, carry):
            base = wid * EPW + blk * B

            @pl.when(blk > 0)
            def _():
                pltpu.make_async_copy(xs, acc.at[didx], ssem).wait()
                pltpu.make_async_copy(xd, acc.at[sidx], ssem).wait()

            pltpu.sync_copy(src_hbm.at[pl.ds(base, B)], sidx)
            pltpu.sync_copy(dst_hbm.at[pl.ds(base, B)], didx)
            cps = [pltpu.async_copy(state_hbm.at[sidx], xs, gsem),
                   pltpu.async_copy(state_hbm.at[didx], xd, gsem),
                   pltpu.async_copy(y_hbm.at[sidx], ys, gsem),
                   pltpu.async_copy(y_hbm.at[didx], yd, gsem)]
            for cp in cps:
                cp.wait()

            for g in range(B // 16):
                rows = g * 16 + lanes
                e_acc = jnp.zeros((16,), jnp.float32)
                for k in range(5):
                    col = jnp.full((16,), k, jnp.int32)
                    a = plsc.load_gather(ys, [rows, col])
                    b = plsc.load_gather(yd, [rows, col])
                    e_acc = e_acc + wk_vecs[k] * jnp.abs(a - b)
                en[pl.ds(g * 16, 16)] = e_acc

            @plsc.parallel_loop(0, B, 1, unroll=4)
            def mbody(r):
                s = plsc.load_gather(en, [jnp.full((16,), r, jnp.int32)])
                for j in range(D // 16):
                    a = xs[r, pl.ds(16 * j, 16)]
                    b = xd[r, pl.ds(16 * j, 16)]
                    m = (a - b) * s
                    xs[r, pl.ds(16 * j, 16)] = m
                    xd[r, pl.ds(16 * j, 16)] = -m

            pltpu.async_copy(xs, acc.at[didx], ssem, add=True)
            pltpu.async_copy(xd, acc.at[sidx], ssem, add=True)
            return carry

        lax.fori_loop(0, NBLK, blk_body, 0)
        pltpu.make_async_copy(xs, acc.at[didx], ssem).wait()
        pltpu.make_async_copy(xd, acc.at[sidx], ssem).wait()
        plsc.subcore_barrier()

        for k in range(_NCHUNK_CEIL):
            chunk = sid + k * NS

            @pl.when(chunk < NROWCHUNKS)
            def _():
                pltpu.sync_copy(acc.at[pl.ds(chunk * B, B)],
                                out_hbm.at[pl.ds(cid * N + chunk * B, B)])

    return pl.kernel(
        body,
        out_type=jax.ShapeDtypeStruct((2 * N, D), jnp.float32),
        mesh=mesh,
        scratch_types=[
            pltpu.VMEM_SHARED((N, D), jnp.float32),
            pltpu.VMEM((B, D), jnp.float32),
            pltpu.VMEM((B, D), jnp.float32),
            pltpu.VMEM((B, YC), jnp.float32),
            pltpu.VMEM((B, YC), jnp.float32),
            pltpu.VMEM((B,), jnp.float32),
            pltpu.VMEM((B,), jnp.int32),
            pltpu.VMEM((B,), jnp.int32),
            pltpu.VMEM((16,), jnp.float32),
            pltpu.SemaphoreType.DMA,
            pltpu.SemaphoreType.DMA,
        ],
        compiler_params=pltpu.CompilerParams(needs_layout_passes=False,
                                             use_tc_tiling_on_sc=False),
    )


RB = 1000


def _proj_body(s_ref, w_ref, y_ref):
    y_ref[...] = jnp.dot(s_ref[...], w_ref[...],
                         preferred_element_type=jnp.float32)


def _update_body(s_ref, a0_ref, a1_ref, w_ref, o_ref, y_ref):
    ns = s_ref[...] - STEP * (a0_ref[...] + a1_ref[...])
    o_ref[...] = ns
    y_ref[...] = jnp.dot(ns, w_ref[...], preferred_element_type=jnp.float32)


def _make_tc_kernels():
    grid = (N // RB,)
    s_spec = pl.BlockSpec((RB, D), lambda i: (i, 0))
    w_spec = pl.BlockSpec((D, YC), lambda i: (0, 0))
    y_spec = pl.BlockSpec((RB, YC), lambda i: (i, 0))
    proj = pl.pallas_call(
        _proj_body,
        grid=grid,
        in_specs=[s_spec, w_spec],
        out_specs=y_spec,
        out_shape=jax.ShapeDtypeStruct((N, YC), jnp.float32),
    )
    a0_spec = pl.BlockSpec((RB, D), lambda i: (i, 0))
    a1_spec = pl.BlockSpec((RB, D), lambda i: (i + N // RB, 0))
    update = pl.pallas_call(
        _update_body,
        grid=grid,
        in_specs=[s_spec, a0_spec, a1_spec, w_spec],
        out_specs=[s_spec, y_spec],
        out_shape=[jax.ShapeDtypeStruct((N, D), jnp.float32),
                   jax.ShapeDtypeStruct((N, YC), jnp.float32)],
    )
    return proj, update


def kernel(x, W, bobot, edge_index):
    w = jax.nn.softmax(bobot)
    w16 = jnp.zeros((16,), jnp.float32).at[:5].set(w)
    Wp = jnp.zeros((D, YC), jnp.float32).at[:, :5].set(W)
    src = edge_index[0]
    dst = edge_index[1]

    sc_step = _sc_step_kernel()
    proj, update = _make_tc_kernels()

    state = x
    Y = proj(state, Wp)
    for _ in range(MAX_ITER):
        acc = sc_step(state, Y, src, dst, w16)
        state, Y = update(state, acc, acc, Wp)
    return state
